# baseline (speedup 1.0000x reference)
import functools

import jax
import jax.numpy as jnp
from jax import lax
from jax.experimental import pallas as pl
from jax.experimental.pallas import tpu as pltpu
from jax.experimental.pallas import tpu_sc as plsc

N = 8192
C = 512
H = 256
NCLS = 2

TR = 256
TC_ = 2048
NI = N // TR
NJ = N // TC_

_DN = (((1,), (1,)), ((), ()))


def _feat_body(x_ref, w1_ref, b1_ref, wroot_ref,
               probs_ref, h_ref, hroot_ref, sqt_ref):
    xb = x_ref[...]
    m = jnp.max(xb, axis=1, keepdims=True)
    e = jnp.exp(xb - m)
    p = e / jnp.sum(e, axis=1, keepdims=True)
    probs_ref[...] = p
    hb = jnp.maximum(
        lax.dot_general(xb, w1_ref[...], _DN,
                        preferred_element_type=jnp.float32) + b1_ref[...],
        0.0)
    h_ref[...] = hb
    hroot_ref[...] = lax.dot_general(hb, wroot_ref[...], _DN,
                                     preferred_element_type=jnp.float32)
    p2 = p * p
    sqt_ref[...] = lax.dot_general(
        jnp.ones((1, C), jnp.float32), p2, _DN,
        preferred_element_type=jnp.float32,
        precision=lax.Precision.HIGHEST)


def _knn_body(pr_ref, pfull_ref, sqt_ref, i0_ref, i1_ref, i2_ref,
              rv1_ref, rc1_ref, rv2_ref, rc2_ref, rv3_ref, rc3_ref):
    prm2 = pr_ref[...] * (-2.0)
    big_i = jnp.int32(2**30)
    inf = jnp.float32(jnp.inf)
    lane8 = lax.broadcasted_iota(jnp.int32, (8, 128), 1)
    NG = TC_ // 128
    for ref in (rv1_ref, rv2_ref, rv3_ref):
        ref[...] = jnp.full((TR, 128), inf, jnp.float32)

    def fold_into(dot, j):
        sqc = sqt_ref[:, pl.ds(j * TC_, TC_)]
        lanej = lane8 + j * TC_
        for r in range(TR // 8):
            rs = slice(r * 8, (r + 1) * 8)
            s = []
            for k in range(0, NG, 2):
                a = dot[rs, k * 128:(k + 1) * 128] \
                    + sqc[:, k * 128:(k + 1) * 128]
                bb = dot[rs, (k + 1) * 128:(k + 2) * 128] \
                    + sqc[:, (k + 1) * 128:(k + 2) * 128]
                le = a <= bb
                s.append((jnp.minimum(a, bb),
                          jnp.where(le, jnp.int32(k), jnp.int32(k + 1)),
                          jnp.maximum(a, bb),
                          jnp.where(le, jnp.int32(k + 1), jnp.int32(k))))
            while len(s) > 1:
                ns = []
                for k in range(0, len(s), 2):
                    u1, gu1, u2, gu2 = s[k]
                    w1, gw1, w2, gw2 = s[k + 1]
                    le1 = u1 <= w1
                    m1 = jnp.minimum(u1, w1)
                    g1 = jnp.where(le1, gu1, gw1)
                    hi = jnp.maximum(u1, w1)
                    gh = jnp.where(le1, gw1, gu1)
                    le2 = u2 <= w2
                    c2 = jnp.minimum(u2, w2)
                    gc2 = jnp.where(le2, gu2, gw2)
                    pick = hi <= c2
                    m2 = jnp.where(pick, hi, c2)
                    g2 = jnp.where(pick, gh, gc2)
                    ns.append((m1, g1, m2, g2))
                s = ns
            gv1, gg1, gv2, gg2 = s[0]
            rv1 = rv1_ref[rs, :]
            rc1 = rc1_ref[rs, :]
            rv2 = rv2_ref[rs, :]
            rc2 = rc2_ref[rs, :]
            rv3 = rv3_ref[rs, :]
            rc3 = rc3_ref[rs, :]
            for bv, bc in ((gv1, gg1 * 128 + lanej), (gv2, gg2 * 128 + lanej)):
                c1 = bv < rv1
                c2 = bv < rv2
                c3 = bv < rv3
                rv3 = jnp.where(c3, jnp.where(c2, rv2, bv), rv3)
                rc3 = jnp.where(c3, jnp.where(c2, rc2, bc), rc3)
                rv2 = jnp.where(c2, jnp.where(c1, rv1, bv), rv2)
                rc2 = jnp.where(c2, jnp.where(c1, rc1, bc), rc2)
                rv1 = jnp.where(c1, bv, rv1)
                rc1 = jnp.where(c1, bc, rc1)
            rv1_ref[rs, :] = rv1
            rc1_ref[rs, :] = rc1
            rv2_ref[rs, :] = rv2
            rc2_ref[rs, :] = rc2
            rv3_ref[rs, :] = rv3
            rc3_ref[rs, :] = rc3

    def chunk(j, dotp):
        dotn = lax.dot_general(prm2,
                               pfull_ref[pl.ds((j + 1) * TC_, TC_), :], _DN,
                               preferred_element_type=jnp.float32)
        fold_into(dotp, j)
        return dotn

    dot0 = lax.dot_general(prm2, pfull_ref[0:TC_, :], _DN,
                           preferred_element_type=jnp.float32)
    dotlast = lax.fori_loop(0, NJ - 1, chunk, dot0)
    fold_into(dotlast, NJ - 1)
    rv1 = rv1_ref[...]
    rc1 = rc1_ref[...]
    rv2 = rv2_ref[...]
    rc2 = rc2_ref[...]
    rv3 = rv3_ref[...]
    rc3 = rc3_ref[...]
    outs = (i0_ref, i1_ref, i2_ref)
    for t in range(3):
        mv = jnp.min(rv1, axis=1, keepdims=True)
        eq = rv1 == mv
        cand = jnp.where(eq, rc1, big_i)
        mi = jnp.min(cand, axis=1, keepdims=True)
        if t < 2:
            hit = cand == mi
            rv1 = jnp.where(hit, rv2, rv1)
            rc1 = jnp.where(hit, rc2, rc1)
            rv2 = jnp.where(hit, rv3, rv2)
            rc2 = jnp.where(hit, rc3, rc2)
            rv3 = jnp.where(hit, inf, rv3)
        outs[t][...] = mi


def _out_body(agg_ref, hroot_ref, wrel_ref, brel_ref, w2_ref, b2_ref,
              out_ref, x1_ref):
    x1 = jnp.maximum(
        lax.dot_general(agg_ref[...], wrel_ref[...], _DN,
                        preferred_element_type=jnp.float32)
        + brel_ref[...] + hroot_ref[...],
        0.0)
    x1_ref[...] = x1
    out_ref[...] = lax.dot_general(x1, w2_ref[...], _DN,
                                   preferred_element_type=jnp.float32) \
        + b2_ref[...]


_NC = 2
_NS = 16
_NW = _NC * _NS
_RPW = N // _NW
_CH = 64


def _gather_sum(h, i0, i1, i2):
    mesh = plsc.VectorSubcoreMesh(core_axis_name="c", subcore_axis_name="s")

    @functools.partial(
        pl.kernel, mesh=mesh,
        out_type=jax.ShapeDtypeStruct((N, H), jnp.float32),
        scratch_types=[
            pltpu.VMEM((_CH,), jnp.int32),
            pltpu.VMEM((_CH,), jnp.int32),
            pltpu.VMEM((_CH,), jnp.int32),
            pltpu.VMEM((_CH, H), jnp.float32),
            pltpu.VMEM((_CH, H), jnp.float32),
            pltpu.VMEM((_CH, H), jnp.float32),
            pltpu.SemaphoreType.DMA,
        ],
    )
    def k(h_hbm, i0_hbm, i1_hbm, i2_hbm, out_hbm,
          x0, x1_, x2, r0, r1, r2, sem):
        wid = lax.axis_index("s") * _NC + lax.axis_index("c")
        base = wid * _RPW

        def chunk(c, carry):
            start = base + c * _CH
            pltpu.sync_copy(i0_hbm.at[pl.ds(start, _CH)], x0)
            pltpu.sync_copy(i1_hbm.at[pl.ds(start, _CH)], x1_)
            pltpu.sync_copy(i2_hbm.at[pl.ds(start, _CH)], x2)
            cp0 = pltpu.async_copy(h_hbm.at[x0], r0, sem)
            cp1 = pltpu.async_copy(h_hbm.at[x1_], r1, sem)
            cp2 = pltpu.async_copy(h_hbm.at[x2], r2, sem)
            cp0.wait()
            cp1.wait()
            cp2.wait()

            def row(r, rc):
                for g in range(H // 16):
                    sl = pl.ds(g * 16, 16)
                    r0[r, sl] = r0[r, sl] + r1[r, sl] + r2[r, sl]
                return rc

            lax.fori_loop(0, _CH, row, 0)
            pltpu.sync_copy(r0, out_hbm.at[pl.ds(start, _CH)])
            return carry

        lax.fori_loop(0, _RPW // _CH, chunk, 0)

    return k(h, i0, i1, i2)


def kernel(x, W1, b1, Wrel, brel, Wroot, W2, b2):
    probs, h, hroot, sqt = pl.pallas_call(
        _feat_body,
        grid=(NI,),
        in_specs=[
            pl.BlockSpec((TR, C), lambda i: (i, 0)),
            pl.BlockSpec((H, C), lambda i: (0, 0)),
            pl.BlockSpec((1, H), lambda i: (0, 0)),
            pl.BlockSpec((H, H), lambda i: (0, 0)),
        ],
        out_specs=[
            pl.BlockSpec((TR, C), lambda i: (i, 0)),
            pl.BlockSpec((TR, H), lambda i: (i, 0)),
            pl.BlockSpec((TR, H), lambda i: (i, 0)),
            pl.BlockSpec((1, TR), lambda i: (0, i)),
        ],
        out_shape=[
            jax.ShapeDtypeStruct((N, C), jnp.float32),
            jax.ShapeDtypeStruct((N, H), jnp.float32),
            jax.ShapeDtypeStruct((N, H), jnp.float32),
            jax.ShapeDtypeStruct((1, N), jnp.float32),
        ],
    )(x, W1, b1.reshape(1, H), Wroot)

    i0, i1, i2 = pl.pallas_call(
        _knn_body,
        grid=(NI,),
        in_specs=[
            pl.BlockSpec((TR, C), lambda i: (i, 0)),
            pl.BlockSpec((N, C), lambda i: (0, 0)),
            pl.BlockSpec((1, N), lambda i: (0, 0)),
        ],
        out_specs=[
            pl.BlockSpec((TR, 1), lambda i: (i, 0)),
            pl.BlockSpec((TR, 1), lambda i: (i, 0)),
            pl.BlockSpec((TR, 1), lambda i: (i, 0)),
        ],
        out_shape=[
            jax.ShapeDtypeStruct((N, 1), jnp.int32),
            jax.ShapeDtypeStruct((N, 1), jnp.int32),
            jax.ShapeDtypeStruct((N, 1), jnp.int32),
        ],
        scratch_shapes=[
            pltpu.VMEM((TR, 128), jnp.float32),
            pltpu.VMEM((TR, 128), jnp.int32),
            pltpu.VMEM((TR, 128), jnp.float32),
            pltpu.VMEM((TR, 128), jnp.int32),
            pltpu.VMEM((TR, 128), jnp.float32),
            pltpu.VMEM((TR, 128), jnp.int32),
        ],
    )(probs, probs, sqt)

    agg = _gather_sum(h, i0.reshape(N), i1.reshape(N), i2.reshape(N))

    out, x1 = pl.pallas_call(
        _out_body,
        grid=(NI,),
        in_specs=[
            pl.BlockSpec((TR, H), lambda i: (i, 0)),
            pl.BlockSpec((TR, H), lambda i: (i, 0)),
            pl.BlockSpec((H, H), lambda i: (0, 0)),
            pl.BlockSpec((1, H), lambda i: (0, 0)),
            pl.BlockSpec((NCLS, H), lambda i: (0, 0)),
            pl.BlockSpec((1, NCLS), lambda i: (0, 0)),
        ],
        out_specs=[
            pl.BlockSpec((TR, NCLS), lambda i: (i, 0)),
            pl.BlockSpec((TR, H), lambda i: (i, 0)),
        ],
        out_shape=[
            jax.ShapeDtypeStruct((N, NCLS), jnp.float32),
            jax.ShapeDtypeStruct((N, H), jnp.float32),
        ],
    )(agg, hroot, Wrel, brel.reshape(1, H), W2, b2.reshape(1, NCLS))

    return out, x1


# speedup vs baseline: 1.2517x; 1.2517x over previous
import functools

import jax
import jax.numpy as jnp
from jax import lax
from jax.experimental import pallas as pl
from jax.experimental.pallas import tpu as pltpu
from jax.experimental.pallas import tpu_sc as plsc

N = 8192
C = 512
H = 256
NCLS = 2

TR = 256
TC_ = 2048
NI = N // TR
NJ = N // TC_

_DN = (((1,), (1,)), ((), ()))


def _feat_body(x_ref, w1_ref, b1_ref, wroot_ref,
               probs_ref, h_ref, hroot_ref, sqt_ref):
    xb = x_ref[...]
    m = jnp.max(xb, axis=1, keepdims=True)
    e = jnp.exp(xb - m)
    p = e / jnp.sum(e, axis=1, keepdims=True)
    probs_ref[...] = p
    hb = jnp.maximum(
        lax.dot_general(xb, w1_ref[...], _DN,
                        preferred_element_type=jnp.float32) + b1_ref[...],
        0.0)
    h_ref[...] = hb
    hroot_ref[...] = lax.dot_general(hb, wroot_ref[...], _DN,
                                     preferred_element_type=jnp.float32)
    p2 = p * p
    sqt_ref[...] = lax.dot_general(
        jnp.ones((1, C), jnp.float32), p2, _DN,
        preferred_element_type=jnp.float32,
        precision=lax.Precision.HIGHEST)


def _knn_body(pr_ref, pfull_ref, sqt_ref, i0_ref, i1_ref, i2_ref,
              rv1_ref, rc1_ref, rv2_ref, rc2_ref, rv3_ref, rc3_ref):
    prm2 = pr_ref[...] * (-2.0)
    big_i = jnp.int32(2**30)
    inf = jnp.float32(jnp.inf)
    lane8 = lax.broadcasted_iota(jnp.int32, (8, 128), 1)
    NG = TC_ // 128
    for ref in (rv1_ref, rv2_ref, rv3_ref):
        ref[...] = jnp.full((TR, 128), inf, jnp.float32)

    def fold_into(dot, j):
        sqc = sqt_ref[:, pl.ds(j * TC_, TC_)]
        lanej = lane8 + j * TC_
        for r in range(TR // 8):
            rs = slice(r * 8, (r + 1) * 8)
            s = []
            for k in range(0, NG, 2):
                a = dot[rs, k * 128:(k + 1) * 128] \
                    + sqc[:, k * 128:(k + 1) * 128]
                bb = dot[rs, (k + 1) * 128:(k + 2) * 128] \
                    + sqc[:, (k + 1) * 128:(k + 2) * 128]
                le = a <= bb
                s.append((jnp.minimum(a, bb),
                          jnp.where(le, jnp.int32(k), jnp.int32(k + 1)),
                          jnp.maximum(a, bb),
                          jnp.where(le, jnp.int32(k + 1), jnp.int32(k))))
            while len(s) > 1:
                ns = []
                for k in range(0, len(s), 2):
                    u1, gu1, u2, gu2 = s[k]
                    w1, gw1, w2, gw2 = s[k + 1]
                    le1 = u1 <= w1
                    m1 = jnp.minimum(u1, w1)
                    g1 = jnp.where(le1, gu1, gw1)
                    hi = jnp.maximum(u1, w1)
                    gh = jnp.where(le1, gw1, gu1)
                    le2 = u2 <= w2
                    c2 = jnp.minimum(u2, w2)
                    gc2 = jnp.where(le2, gu2, gw2)
                    pick = hi <= c2
                    m2 = jnp.where(pick, hi, c2)
                    g2 = jnp.where(pick, gh, gc2)
                    ns.append((m1, g1, m2, g2))
                s = ns
            gv1, gg1, gv2, gg2 = s[0]
            rv1 = rv1_ref[rs, :]
            rc1 = rc1_ref[rs, :]
            rv2 = rv2_ref[rs, :]
            rc2 = rc2_ref[rs, :]
            rv3 = rv3_ref[rs, :]
            rc3 = rc3_ref[rs, :]
            for bv, bc in ((gv1, gg1 * 128 + lanej), (gv2, gg2 * 128 + lanej)):
                c1 = bv < rv1
                c2 = bv < rv2
                c3 = bv < rv3
                rv3 = jnp.where(c3, jnp.where(c2, rv2, bv), rv3)
                rc3 = jnp.where(c3, jnp.where(c2, rc2, bc), rc3)
                rv2 = jnp.where(c2, jnp.where(c1, rv1, bv), rv2)
                rc2 = jnp.where(c2, jnp.where(c1, rc1, bc), rc2)
                rv1 = jnp.where(c1, bv, rv1)
                rc1 = jnp.where(c1, bc, rc1)
            rv1_ref[rs, :] = rv1
            rc1_ref[rs, :] = rc1
            rv2_ref[rs, :] = rv2
            rc2_ref[rs, :] = rc2
            rv3_ref[rs, :] = rv3
            rc3_ref[rs, :] = rc3

    for j in range(NJ):
        dot = lax.dot_general(prm2, pfull_ref[j * TC_:(j + 1) * TC_, :], _DN,
                              preferred_element_type=jnp.float32)
        fold_into(dot, j)
    rv1 = rv1_ref[...]
    rc1 = rc1_ref[...]
    rv2 = rv2_ref[...]
    rc2 = rc2_ref[...]
    rv3 = rv3_ref[...]
    rc3 = rc3_ref[...]
    outs = (i0_ref, i1_ref, i2_ref)
    for t in range(3):
        mv = jnp.min(rv1, axis=1, keepdims=True)
        eq = rv1 == mv
        cand = jnp.where(eq, rc1, big_i)
        mi = jnp.min(cand, axis=1, keepdims=True)
        if t < 2:
            hit = cand == mi
            rv1 = jnp.where(hit, rv2, rv1)
            rc1 = jnp.where(hit, rc2, rc1)
            rv2 = jnp.where(hit, rv3, rv2)
            rc2 = jnp.where(hit, rc3, rc2)
            rv3 = jnp.where(hit, inf, rv3)
        outs[t][...] = mi


def _out_body(agg_ref, hroot_ref, wrel_ref, brel_ref, w2_ref, b2_ref,
              out_ref, x1_ref):
    x1 = jnp.maximum(
        lax.dot_general(agg_ref[...], wrel_ref[...], _DN,
                        preferred_element_type=jnp.float32)
        + brel_ref[...] + hroot_ref[...],
        0.0)
    x1_ref[...] = x1
    out_ref[...] = lax.dot_general(x1, w2_ref[...], _DN,
                                   preferred_element_type=jnp.float32) \
        + b2_ref[...]


_NC = 2
_NS = 16
_NW = _NC * _NS
_RPW = N // _NW
_CH = 64


def _gather_sum(h, i0, i1, i2):
    mesh = plsc.VectorSubcoreMesh(core_axis_name="c", subcore_axis_name="s")

    @functools.partial(
        pl.kernel, mesh=mesh,
        out_type=jax.ShapeDtypeStruct((N, H), jnp.float32),
        scratch_types=[
            pltpu.VMEM((_CH,), jnp.int32),
            pltpu.VMEM((_CH,), jnp.int32),
            pltpu.VMEM((_CH,), jnp.int32),
            pltpu.VMEM((_CH, H), jnp.float32),
            pltpu.VMEM((_CH, H), jnp.float32),
            pltpu.VMEM((_CH, H), jnp.float32),
            pltpu.SemaphoreType.DMA,
        ],
    )
    def k(h_hbm, i0_hbm, i1_hbm, i2_hbm, out_hbm,
          x0, x1_, x2, r0, r1, r2, sem):
        wid = lax.axis_index("s") * _NC + lax.axis_index("c")
        base = wid * _RPW

        def chunk(c, carry):
            start = base + c * _CH
            pltpu.sync_copy(i0_hbm.at[pl.ds(start, _CH)], x0)
            pltpu.sync_copy(i1_hbm.at[pl.ds(start, _CH)], x1_)
            pltpu.sync_copy(i2_hbm.at[pl.ds(start, _CH)], x2)
            cp0 = pltpu.async_copy(h_hbm.at[x0], r0, sem)
            cp1 = pltpu.async_copy(h_hbm.at[x1_], r1, sem)
            cp2 = pltpu.async_copy(h_hbm.at[x2], r2, sem)
            cp0.wait()
            cp1.wait()
            cp2.wait()

            def row(r, rc):
                for g in range(H // 16):
                    sl = pl.ds(g * 16, 16)
                    r0[r, sl] = r0[r, sl] + r1[r, sl] + r2[r, sl]
                return rc

            lax.fori_loop(0, _CH, row, 0)
            pltpu.sync_copy(r0, out_hbm.at[pl.ds(start, _CH)])
            return carry

        lax.fori_loop(0, _RPW // _CH, chunk, 0)

    return k(h, i0, i1, i2)


def kernel(x, W1, b1, Wrel, brel, Wroot, W2, b2):
    probs, h, hroot, sqt = pl.pallas_call(
        _feat_body,
        grid=(NI,),
        in_specs=[
            pl.BlockSpec((TR, C), lambda i: (i, 0)),
            pl.BlockSpec((H, C), lambda i: (0, 0)),
            pl.BlockSpec((1, H), lambda i: (0, 0)),
            pl.BlockSpec((H, H), lambda i: (0, 0)),
        ],
        out_specs=[
            pl.BlockSpec((TR, C), lambda i: (i, 0)),
            pl.BlockSpec((TR, H), lambda i: (i, 0)),
            pl.BlockSpec((TR, H), lambda i: (i, 0)),
            pl.BlockSpec((1, TR), lambda i: (0, i)),
        ],
        out_shape=[
            jax.ShapeDtypeStruct((N, C), jnp.float32),
            jax.ShapeDtypeStruct((N, H), jnp.float32),
            jax.ShapeDtypeStruct((N, H), jnp.float32),
            jax.ShapeDtypeStruct((1, N), jnp.float32),
        ],
    )(x, W1, b1.reshape(1, H), Wroot)

    i0, i1, i2 = pl.pallas_call(
        _knn_body,
        grid=(NI,),
        in_specs=[
            pl.BlockSpec((TR, C), lambda i: (i, 0)),
            pl.BlockSpec((N, C), lambda i: (0, 0)),
            pl.BlockSpec((1, N), lambda i: (0, 0)),
        ],
        out_specs=[
            pl.BlockSpec((TR, 1), lambda i: (i, 0)),
            pl.BlockSpec((TR, 1), lambda i: (i, 0)),
            pl.BlockSpec((TR, 1), lambda i: (i, 0)),
        ],
        out_shape=[
            jax.ShapeDtypeStruct((N, 1), jnp.int32),
            jax.ShapeDtypeStruct((N, 1), jnp.int32),
            jax.ShapeDtypeStruct((N, 1), jnp.int32),
        ],
        scratch_shapes=[
            pltpu.VMEM((TR, 128), jnp.float32),
            pltpu.VMEM((TR, 128), jnp.int32),
            pltpu.VMEM((TR, 128), jnp.float32),
            pltpu.VMEM((TR, 128), jnp.int32),
            pltpu.VMEM((TR, 128), jnp.float32),
            pltpu.VMEM((TR, 128), jnp.int32),
        ],
    )(probs, probs, sqt)

    agg = _gather_sum(h, i0.reshape(N), i1.reshape(N), i2.reshape(N))

    out, x1 = pl.pallas_call(
        _out_body,
        grid=(NI,),
        in_specs=[
            pl.BlockSpec((TR, H), lambda i: (i, 0)),
            pl.BlockSpec((TR, H), lambda i: (i, 0)),
            pl.BlockSpec((H, H), lambda i: (0, 0)),
            pl.BlockSpec((1, H), lambda i: (0, 0)),
            pl.BlockSpec((NCLS, H), lambda i: (0, 0)),
            pl.BlockSpec((1, NCLS), lambda i: (0, 0)),
        ],
        out_specs=[
            pl.BlockSpec((TR, NCLS), lambda i: (i, 0)),
            pl.BlockSpec((TR, H), lambda i: (i, 0)),
        ],
        out_shape=[
            jax.ShapeDtypeStruct((N, NCLS), jnp.float32),
            jax.ShapeDtypeStruct((N, H), jnp.float32),
        ],
    )(agg, hroot, Wrel, brel.reshape(1, H), W2, b2.reshape(1, NCLS))

    return out, x1


# speedup vs baseline: 1.2661x; 1.0115x over previous
import functools

import jax
import jax.numpy as jnp
from jax import lax
from jax.experimental import pallas as pl
from jax.experimental.pallas import tpu as pltpu
from jax.experimental.pallas import tpu_sc as plsc

N = 8192
C = 512
H = 256
NCLS = 2

TR = 256
TC_ = 2048
NI = N // TR
NJ = N // TC_

_DN = (((1,), (1,)), ((), ()))


def _feat_body(x_ref, w1_ref, b1_ref, wroot_ref,
               probs_ref, h_ref, hroot_ref, sqt_ref):
    xb = x_ref[...]
    m = jnp.max(xb, axis=1, keepdims=True)
    e = jnp.exp(xb - m)
    p = e / jnp.sum(e, axis=1, keepdims=True)
    probs_ref[...] = p
    hb = jnp.maximum(
        lax.dot_general(xb, w1_ref[...], _DN,
                        preferred_element_type=jnp.float32) + b1_ref[...],
        0.0)
    h_ref[...] = hb
    hroot_ref[...] = lax.dot_general(hb, wroot_ref[...], _DN,
                                     preferred_element_type=jnp.float32)
    p2 = p * p
    sqt_ref[...] = lax.dot_general(
        jnp.ones((1, C), jnp.float32), p2, _DN,
        preferred_element_type=jnp.float32,
        precision=lax.Precision.HIGHEST)


def _knn_body(pr_ref, pfull_ref, sqt_ref, i0_ref, i1_ref, i2_ref,
              rv1_ref, rc1_ref, rv2_ref, rc2_ref, rv3_ref, rc3_ref):
    prm2 = pr_ref[...] * (-2.0)
    big_i = jnp.int32(2**30)
    inf = jnp.float32(jnp.inf)
    lane8 = lax.broadcasted_iota(jnp.int32, (8, 128), 1)
    NG = TC_ // 128
    dots = [lax.dot_general(prm2, pfull_ref[j * TC_:(j + 1) * TC_, :], _DN,
                            preferred_element_type=jnp.float32)
            for j in range(NJ)]
    sqb = [jnp.broadcast_to(sqt_ref[:, c * 128:(c + 1) * 128], (8, 128))
           for c in range(N // 128)]

    for r in range(TR // 8):
        rs = slice(r * 8, (r + 1) * 8)
        rv1 = rc1 = rv2 = rc2 = rv3 = rc3 = None
        for j in range(NJ):
            s = []
            for k in range(0, NG, 2):
                g0 = (j * NG + k) * 128
                g1 = g0 + 128
                a = dots[j][rs, k * 128:(k + 1) * 128] + sqb[j * NG + k]
                bb = dots[j][rs, (k + 1) * 128:(k + 2) * 128] \
                    + sqb[j * NG + k + 1]
                le = a <= bb
                s.append((jnp.minimum(a, bb),
                          jnp.where(le, jnp.int32(g0), jnp.int32(g1)),
                          jnp.maximum(a, bb),
                          jnp.where(le, jnp.int32(g1), jnp.int32(g0))))
            while len(s) > 1:
                ns = []
                for k in range(0, len(s), 2):
                    u1, gu1, u2, gu2 = s[k]
                    w1, gw1, w2, gw2 = s[k + 1]
                    le1 = u1 <= w1
                    m1 = jnp.minimum(u1, w1)
                    g1 = jnp.where(le1, gu1, gw1)
                    hi = jnp.maximum(u1, w1)
                    gh = jnp.where(le1, gw1, gu1)
                    le2 = u2 <= w2
                    c2 = jnp.minimum(u2, w2)
                    gc2 = jnp.where(le2, gu2, gw2)
                    pick = hi <= c2
                    m2 = jnp.where(pick, hi, c2)
                    g2 = jnp.where(pick, gh, gc2)
                    ns.append((m1, g1, m2, g2))
                s = ns
            gv1, gg1, gv2, gg2 = s[0]
            bc1 = gg1 + lane8
            bc2 = gg2 + lane8
            if rv1 is None:
                rv1, rc1, rv2, rc2 = gv1, bc1, gv2, bc2
                rv3 = jnp.full((8, 128), inf, jnp.float32)
                rc3 = bc1
                continue
            c1 = gv1 < rv1
            c2 = gv1 < rv2
            c3 = gv1 < rv3
            rv3 = jnp.where(c3, jnp.where(c2, rv2, gv1), rv3)
            rc3 = jnp.where(c3, jnp.where(c2, rc2, bc1), rc3)
            rv2 = jnp.where(c2, jnp.where(c1, rv1, gv1), rv2)
            rc2 = jnp.where(c2, jnp.where(c1, rc1, bc1), rc2)
            rv1 = jnp.where(c1, gv1, rv1)
            rc1 = jnp.where(c1, bc1, rc1)
            c2 = gv2 < rv2
            c3 = gv2 < rv3
            rv3 = jnp.where(c3, jnp.where(c2, rv2, gv2), rv3)
            rc3 = jnp.where(c3, jnp.where(c2, rc2, bc2), rc3)
            rv2 = jnp.where(c2, gv2, rv2)
            rc2 = jnp.where(c2, bc2, rc2)
        rv1_ref[rs, :] = rv1
        rc1_ref[rs, :] = rc1
        rv2_ref[rs, :] = rv2
        rc2_ref[rs, :] = rc2
        rv3_ref[rs, :] = rv3
        rc3_ref[rs, :] = rc3
    rv1 = rv1_ref[...]
    rc1 = rc1_ref[...]
    rv2 = rv2_ref[...]
    rc2 = rc2_ref[...]
    rv3 = rv3_ref[...]
    rc3 = rc3_ref[...]
    outs = (i0_ref, i1_ref, i2_ref)
    for t in range(3):
        mv = jnp.min(rv1, axis=1, keepdims=True)
        eq = rv1 == mv
        cand = jnp.where(eq, rc1, big_i)
        mi = jnp.min(cand, axis=1, keepdims=True)
        if t < 2:
            hit = cand == mi
            rv1 = jnp.where(hit, rv2, rv1)
            rc1 = jnp.where(hit, rc2, rc1)
            rv2 = jnp.where(hit, rv3, rv2)
            rc2 = jnp.where(hit, rc3, rc2)
            rv3 = jnp.where(hit, inf, rv3)
        outs[t][...] = mi


def _out_body(agg_ref, hroot_ref, wrel_ref, brel_ref, w2_ref, b2_ref,
              out_ref, x1_ref):
    x1 = jnp.maximum(
        lax.dot_general(agg_ref[...], wrel_ref[...], _DN,
                        preferred_element_type=jnp.float32)
        + brel_ref[...] + hroot_ref[...],
        0.0)
    x1_ref[...] = x1
    out_ref[...] = lax.dot_general(x1, w2_ref[...], _DN,
                                   preferred_element_type=jnp.float32) \
        + b2_ref[...]


_NC = 2
_NS = 16
_NW = _NC * _NS
_RPW = N // _NW
_CH = 64


def _gather_sum(h, i0, i1, i2):
    mesh = plsc.VectorSubcoreMesh(core_axis_name="c", subcore_axis_name="s")

    @functools.partial(
        pl.kernel, mesh=mesh,
        out_type=jax.ShapeDtypeStruct((N, H), jnp.float32),
        scratch_types=[
            pltpu.VMEM((_CH,), jnp.int32),
            pltpu.VMEM((_CH,), jnp.int32),
            pltpu.VMEM((_CH,), jnp.int32),
            pltpu.VMEM((_CH, H), jnp.float32),
            pltpu.VMEM((_CH, H), jnp.float32),
            pltpu.VMEM((_CH, H), jnp.float32),
            pltpu.SemaphoreType.DMA,
        ],
    )
    def k(h_hbm, i0_hbm, i1_hbm, i2_hbm, out_hbm,
          x0, x1_, x2, r0, r1, r2, sem):
        wid = lax.axis_index("s") * _NC + lax.axis_index("c")
        base = wid * _RPW

        def chunk(c, carry):
            start = base + c * _CH
            pltpu.sync_copy(i0_hbm.at[pl.ds(start, _CH)], x0)
            pltpu.sync_copy(i1_hbm.at[pl.ds(start, _CH)], x1_)
            pltpu.sync_copy(i2_hbm.at[pl.ds(start, _CH)], x2)
            cp0 = pltpu.async_copy(h_hbm.at[x0], r0, sem)
            cp1 = pltpu.async_copy(h_hbm.at[x1_], r1, sem)
            cp2 = pltpu.async_copy(h_hbm.at[x2], r2, sem)
            cp0.wait()
            cp1.wait()
            cp2.wait()

            def row(r, rc):
                for g in range(H // 16):
                    sl = pl.ds(g * 16, 16)
                    r0[r, sl] = r0[r, sl] + r1[r, sl] + r2[r, sl]
                return rc

            lax.fori_loop(0, _CH, row, 0)
            pltpu.sync_copy(r0, out_hbm.at[pl.ds(start, _CH)])
            return carry

        lax.fori_loop(0, _RPW // _CH, chunk, 0)

    return k(h, i0, i1, i2)


def kernel(x, W1, b1, Wrel, brel, Wroot, W2, b2):
    probs, h, hroot, sqt = pl.pallas_call(
        _feat_body,
        grid=(NI,),
        in_specs=[
            pl.BlockSpec((TR, C), lambda i: (i, 0)),
            pl.BlockSpec((H, C), lambda i: (0, 0)),
            pl.BlockSpec((1, H), lambda i: (0, 0)),
            pl.BlockSpec((H, H), lambda i: (0, 0)),
        ],
        out_specs=[
            pl.BlockSpec((TR, C), lambda i: (i, 0)),
            pl.BlockSpec((TR, H), lambda i: (i, 0)),
            pl.BlockSpec((TR, H), lambda i: (i, 0)),
            pl.BlockSpec((1, TR), lambda i: (0, i)),
        ],
        out_shape=[
            jax.ShapeDtypeStruct((N, C), jnp.float32),
            jax.ShapeDtypeStruct((N, H), jnp.float32),
            jax.ShapeDtypeStruct((N, H), jnp.float32),
            jax.ShapeDtypeStruct((1, N), jnp.float32),
        ],
    )(x, W1, b1.reshape(1, H), Wroot)

    i0, i1, i2 = pl.pallas_call(
        _knn_body,
        grid=(NI,),
        in_specs=[
            pl.BlockSpec((TR, C), lambda i: (i, 0)),
            pl.BlockSpec((N, C), lambda i: (0, 0)),
            pl.BlockSpec((1, N), lambda i: (0, 0)),
        ],
        out_specs=[
            pl.BlockSpec((TR, 1), lambda i: (i, 0)),
            pl.BlockSpec((TR, 1), lambda i: (i, 0)),
            pl.BlockSpec((TR, 1), lambda i: (i, 0)),
        ],
        out_shape=[
            jax.ShapeDtypeStruct((N, 1), jnp.int32),
            jax.ShapeDtypeStruct((N, 1), jnp.int32),
            jax.ShapeDtypeStruct((N, 1), jnp.int32),
        ],
        scratch_shapes=[
            pltpu.VMEM((TR, 128), jnp.float32),
            pltpu.VMEM((TR, 128), jnp.int32),
            pltpu.VMEM((TR, 128), jnp.float32),
            pltpu.VMEM((TR, 128), jnp.int32),
            pltpu.VMEM((TR, 128), jnp.float32),
            pltpu.VMEM((TR, 128), jnp.int32),
        ],
    )(probs, probs, sqt)

    agg = _gather_sum(h, i0.reshape(N), i1.reshape(N), i2.reshape(N))

    out, x1 = pl.pallas_call(
        _out_body,
        grid=(NI,),
        in_specs=[
            pl.BlockSpec((TR, H), lambda i: (i, 0)),
            pl.BlockSpec((TR, H), lambda i: (i, 0)),
            pl.BlockSpec((H, H), lambda i: (0, 0)),
            pl.BlockSpec((1, H), lambda i: (0, 0)),
            pl.BlockSpec((NCLS, H), lambda i: (0, 0)),
            pl.BlockSpec((1, NCLS), lambda i: (0, 0)),
        ],
        out_specs=[
            pl.BlockSpec((TR, NCLS), lambda i: (i, 0)),
            pl.BlockSpec((TR, H), lambda i: (i, 0)),
        ],
        out_shape=[
            jax.ShapeDtypeStruct((N, NCLS), jnp.float32),
            jax.ShapeDtypeStruct((N, H), jnp.float32),
        ],
    )(agg, hroot, Wrel, brel.reshape(1, H), W2, b2.reshape(1, NCLS))

    return out, x1
